# baseline (speedup 1.0000x reference)
import functools
import jax
import jax.numpy as jnp
from jax import lax
from jax.experimental import pallas as pl
from jax.experimental.pallas import tpu as pltpu
from jax.experimental.pallas import tpu_sc as plsc


def _sigmoid(x):
    return 0.5 * jnp.tanh(0.5 * x) + 0.5


def _tc_body(xyz_ref, rgb_ref, xyz_out, rgb_out):
    xyz_out[...] = xyz_ref[...]
    rgb_out[...] = _sigmoid(rgb_ref[...])


NW = 32
CHUNK = 16000
NCHUNK = 125


def _sc_body(opa_hbm, scl_hbm, opa_out, scl_out, buf_a, buf_b, sem_a, sem_b):
    w = lax.axis_index("s") * 2 + lax.axis_index("c")

    def process(chunk_id, in_hbm, out_hbm, buf, sem):
        base = chunk_id * CHUNK
        pltpu.async_copy(in_hbm.at[0, pl.ds(base, CHUNK)], buf, sem).wait()

        def body(j, _):
            x = buf[pl.ds(j * 16, 16)]
            y = 1.0 / (1.0 + jnp.exp(-x))
            buf[pl.ds(j * 16, 16)] = y
            return 0

        lax.fori_loop(0, CHUNK // 16, body, 0)
        pltpu.async_copy(buf, out_hbm.at[0, pl.ds(base, CHUNK)], sem).wait()

    def process_exp(chunk_id, in_hbm, out_hbm, buf, sem):
        base = chunk_id * CHUNK
        pltpu.async_copy(in_hbm.at[0, pl.ds(base, CHUNK)], buf, sem).wait()

        def body(j, _):
            x = buf[pl.ds(j * 16, 16)]
            buf[pl.ds(j * 16, 16)] = jnp.exp(x)
            return 0

        lax.fori_loop(0, CHUNK // 16, body, 0)
        pltpu.async_copy(buf, out_hbm.at[0, pl.ds(base, CHUNK)], sem).wait()

    def loop(i, _):
        chunk = w + NW * i

        @pl.when(chunk < NCHUNK)
        def _():
            process(chunk, opa_hbm, opa_out, buf_a, sem_a)
            process_exp(chunk, scl_hbm, scl_out, buf_b, sem_b)

        return 0

    lax.fori_loop(0, (NCHUNK + NW - 1) // NW, loop, 0)


def kernel(xyz_raw, rgb_raw, opacity_raw, scale_raw):
    n = rgb_raw.shape[0]
    xyzT = xyz_raw.T
    rgbT = rgb_raw.T
    opaT = opacity_raw.T
    sclT = scale_raw.T

    C = 262144
    grid = (pl.cdiv(n, C),)
    s3 = pl.BlockSpec((3, C), lambda i: (0, i))
    xyz_o, rgb_o = pl.pallas_call(
        _tc_body,
        grid=grid,
        in_specs=[s3, s3],
        out_specs=[s3, s3],
        out_shape=[
            jax.ShapeDtypeStruct((3, n), jnp.float32),
            jax.ShapeDtypeStruct((3, n), jnp.float32),
        ],
        compiler_params=pltpu.CompilerParams(
            dimension_semantics=("arbitrary",),
        ),
    )(xyzT, rgbT)

    mesh = plsc.VectorSubcoreMesh(core_axis_name="c", subcore_axis_name="s")
    sc_fn = functools.partial(
        pl.kernel,
        out_type=[
            jax.ShapeDtypeStruct((1, n), jnp.float32),
            jax.ShapeDtypeStruct((1, n), jnp.float32),
        ],
        mesh=mesh,
        scratch_types=[
            pltpu.VMEM((CHUNK,), jnp.float32),
            pltpu.VMEM((CHUNK,), jnp.float32),
            pltpu.SemaphoreType.DMA,
            pltpu.SemaphoreType.DMA,
        ],
    )(_sc_body)
    opa_o, scl_o = sc_fn(opaT, sclT)

    return (xyz_o.T, rgb_o.T, opa_o.T, scl_o.T)


# speedup vs baseline: 2.4243x; 2.4243x over previous
import functools
import jax
import jax.numpy as jnp
from jax import lax
from jax.experimental import pallas as pl
from jax.experimental.pallas import tpu as pltpu
from jax.experimental.pallas import tpu_sc as plsc


def _sigmoid(x):
    return 0.5 * jnp.tanh(0.5 * x) + 0.5


def _tc_body(xyz_ref, rgb_ref, xyz_out, rgb_out):
    xyz_out[...] = xyz_ref[...]
    rgb_out[...] = _sigmoid(rgb_ref[...])


NW = 32
CHUNK = 16000
NCHUNK = 125


def _sc_body(opa_hbm, scl_hbm, opa_out, scl_out, buf_a, buf_b, sem_a, sem_b):
    w = lax.axis_index("s") * 2 + lax.axis_index("c")

    def process(chunk_id, in_hbm, out_hbm, buf, sem):
        base = chunk_id * CHUNK
        pltpu.async_copy(in_hbm.at[0, pl.ds(base, CHUNK)], buf, sem).wait()

        @plsc.parallel_loop(0, CHUNK, step=16, unroll=8)
        def _(j):
            x = buf[pl.ds(j, 16)]
            buf[pl.ds(j, 16)] = 1.0 / (1.0 + jnp.exp(-x))

        pltpu.async_copy(buf, out_hbm.at[0, pl.ds(base, CHUNK)], sem).wait()

    def process_exp(chunk_id, in_hbm, out_hbm, buf, sem):
        base = chunk_id * CHUNK
        pltpu.async_copy(in_hbm.at[0, pl.ds(base, CHUNK)], buf, sem).wait()

        @plsc.parallel_loop(0, CHUNK, step=16, unroll=8)
        def _(j):
            x = buf[pl.ds(j, 16)]
            buf[pl.ds(j, 16)] = jnp.exp(x)

        pltpu.async_copy(buf, out_hbm.at[0, pl.ds(base, CHUNK)], sem).wait()

    def loop(i, _):
        chunk = w + NW * i

        @pl.when(chunk < NCHUNK)
        def _():
            process(chunk, opa_hbm, opa_out, buf_a, sem_a)
            process_exp(chunk, scl_hbm, scl_out, buf_b, sem_b)

        return 0

    lax.fori_loop(0, (NCHUNK + NW - 1) // NW, loop, 0)


def kernel(xyz_raw, rgb_raw, opacity_raw, scale_raw):
    n = rgb_raw.shape[0]
    xyzT = xyz_raw.T
    rgbT = rgb_raw.T
    opaT = opacity_raw.T
    sclT = scale_raw.T

    C = 262144
    grid = (pl.cdiv(n, C),)
    s3 = pl.BlockSpec((3, C), lambda i: (0, i))
    xyz_o, rgb_o = pl.pallas_call(
        _tc_body,
        grid=grid,
        in_specs=[s3, s3],
        out_specs=[s3, s3],
        out_shape=[
            jax.ShapeDtypeStruct((3, n), jnp.float32),
            jax.ShapeDtypeStruct((3, n), jnp.float32),
        ],
        compiler_params=pltpu.CompilerParams(
            dimension_semantics=("arbitrary",),
        ),
    )(xyzT, rgbT)

    mesh = plsc.VectorSubcoreMesh(core_axis_name="c", subcore_axis_name="s")
    sc_fn = functools.partial(
        pl.kernel,
        out_type=[
            jax.ShapeDtypeStruct((1, n), jnp.float32),
            jax.ShapeDtypeStruct((1, n), jnp.float32),
        ],
        mesh=mesh,
        scratch_types=[
            pltpu.VMEM((CHUNK,), jnp.float32),
            pltpu.VMEM((CHUNK,), jnp.float32),
            pltpu.SemaphoreType.DMA,
            pltpu.SemaphoreType.DMA,
        ],
    )(_sc_body)
    opa_o, scl_o = sc_fn(opaT, sclT)

    return (xyz_o.T, rgb_o.T, opa_o.T, scl_o.T)


# speedup vs baseline: 3.2826x; 1.3541x over previous
import jax
import jax.numpy as jnp
from jax.experimental import pallas as pl
from jax.experimental.pallas import tpu as pltpu


def _sigmoid(x):
    return 0.5 * jnp.tanh(0.5 * x) + 0.5


def _act_body(xyz_ref, rgb_ref, opa_ref, scl_ref,
              xyz_out, rgb_out, opa_out, scl_out):
    xyz_out[...] = xyz_ref[...]
    rgb_out[...] = _sigmoid(rgb_ref[...])
    opa_out[...] = _sigmoid(opa_ref[...])
    scl_out[...] = jnp.exp(scl_ref[...])


def kernel(xyz_raw, rgb_raw, opacity_raw, scale_raw):
    n = rgb_raw.shape[0]
    xyzT = xyz_raw.T
    rgbT = rgb_raw.T
    opaT = opacity_raw.T
    sclT = scale_raw.T

    C = 327680
    grid = (pl.cdiv(n, C),)
    s3 = pl.BlockSpec((3, C), lambda i: (0, i))
    s1 = pl.BlockSpec((1, C), lambda i: (0, i))
    xyz_o, rgb_o, opa_o, scl_o = pl.pallas_call(
        _act_body,
        grid=grid,
        in_specs=[s3, s3, s1, s1],
        out_specs=[s3, s3, s1, s1],
        out_shape=[
            jax.ShapeDtypeStruct((3, n), jnp.float32),
            jax.ShapeDtypeStruct((3, n), jnp.float32),
            jax.ShapeDtypeStruct((1, n), jnp.float32),
            jax.ShapeDtypeStruct((1, n), jnp.float32),
        ],
        compiler_params=pltpu.CompilerParams(
            dimension_semantics=("arbitrary",),
        ),
    )(xyzT, rgbT, opaT, sclT)

    return (xyz_o.T, rgb_o.T, opa_o.T, scl_o.T)
